# baseline (speedup 1.0000x reference)
import numpy as np
import jax
import jax.numpy as jnp
from jax.experimental import pallas as pl
from jax.experimental.pallas import tpu as pltpu

_B, _N, _C = 2, 8192, 32
_NPOINT = 1024
_NSAMPLE = 32
_RADIUS2 = np.float32(0.8 * 0.8)
_CIN = _C + 3
_CMID, _COUT = 64, 64
_ROWS, _LANES = 64, 128
_PROWS = _NPOINT // _LANES
_PBLK = 128

_HIGHEST = jax.lax.Precision.HIGHEST


_SL = 8
_NSLICE = _ROWS // _SL


def _fps_kernel(xyz_ref, xyzrow_ref, pos_ref, out_ref, dists_ref):
    def lexmax(a, b):
        take_b = jnp.logical_or(
            b[0] > a[0], jnp.logical_and(b[0] == a[0], b[1] < a[1]))
        return tuple(jnp.where(take_b, bb, aa) for aa, bb in zip(a, b))

    def lane_phase(w, shifts):
        rolled = [tuple(pltpu.roll(t, sh, 1) for t in w) for sh in shifts]
        for r in rolled:
            w = lexmax(w, r)
        return w

    def body(i, st):
        ws = []
        for b in range(_B):
            cx, cy, cz = st[b]
            out_ref[b, pl.ds(i, 1), :] = jnp.concatenate(
                [cx[0:1, 0:1], cy[0:1, 0:1], cz[0:1, 0:1]], axis=1)
            w = None
            for j in range(_NSLICE):
                sl = slice(j * _SL, (j + 1) * _SL)
                xj = xyz_ref[b, 0, sl, :]
                yj = xyz_ref[b, 1, sl, :]
                zj = xyz_ref[b, 2, sl, :]
                dx = xj - cx
                dy = yj - cy
                dz = zj - cz
                d = dx * dx + dy * dy + dz * dz
                dj = jnp.minimum(dists_ref[b, sl, :], d)
                dists_ref[b, sl, :] = dj
                cand = (dj, pos_ref[sl, :], xj, yj, zj)
                w = cand if w is None else lexmax(w, cand)
            for sh in (4, 2, 1):
                w = lexmax(w, tuple(pltpu.roll(t, sh, 0) for t in w))
            ws.append(w)
        ws = [lane_phase(w, (1, 2, 3)) for w in ws]
        ws = [lane_phase(w, (4, 8, 12)) for w in ws]
        ws = [lane_phase(w, (16, 32, 48)) for w in ws]
        ws = [lane_phase(w, (64,)) for w in ws]
        return tuple((w[2], w[3], w[4]) for w in ws)

    dists_ref[...] = jnp.full((_B, _ROWS, _LANES), 1e10, jnp.float32)
    init = []
    for b in range(_B):
        c0 = [jnp.full((_SL, _LANES), xyzrow_ref[b, 0, k], jnp.float32)
              for k in range(3)]
        init.append((c0[0], c0[1], c0[2]))
    jax.lax.fori_loop(0, _NPOINT, body, tuple(init))


def _group_mlp_kernel(xyzt_ref, nxyz_ref, src_ref, w1_ref, b1_ref,
                      w2_ref, b2_ref, out_ref):
    xs = xyzt_ref[0]
    cx = nxyz_ref[0]

    d2 = None
    for k in range(3):
        diff = cx[:, k:k + 1] - xs[k:k + 1, :]
        sq = diff * diff
        d2 = sq if d2 is None else d2 + sq
    mask = d2 < _RADIUS2
    mi = mask.astype(jnp.int32)

    c = mi
    sh = 1
    while sh < _N:
        c = c + jnp.concatenate(
            [jnp.zeros((_PBLK, sh), jnp.int32), c[:, :_N - sh]], axis=1)
        sh *= 2
    rank = c - mi
    cnt = c[:, _N - 1:_N]

    srcf = src_ref[0]
    hi = srcf.astype(jnp.bfloat16)
    r1 = srcf - hi.astype(jnp.float32)
    lo = r1.astype(jnp.bfloat16)
    lo2 = (r1 - lo.astype(jnp.float32)).astype(jnp.bfloat16)

    nxyz_pad = jnp.concatenate(
        [cx, jnp.zeros((_PBLK, _CIN - 3), jnp.float32)], axis=1)
    w1 = w1_ref[...]
    b1 = b1_ref[...]
    w2 = w2_ref[...]
    b2 = b2_ref[...]

    def slot(j, pooled):
        eqb = jnp.logical_and(mask, rank == j).astype(jnp.bfloat16)
        g = (jnp.dot(eqb, hi, preferred_element_type=jnp.float32)
             + jnp.dot(eqb, lo, preferred_element_type=jnp.float32)
             + jnp.dot(eqb, lo2, preferred_element_type=jnp.float32))
        u = g - nxyz_pad
        h1 = jnp.maximum(
            jnp.dot(u, w1, preferred_element_type=jnp.float32,
                    precision=_HIGHEST) + b1, 0.0)
        h2 = jnp.maximum(
            jnp.dot(h1, w2, preferred_element_type=jnp.float32,
                    precision=_HIGHEST) + b2, 0.0)
        h2 = jnp.where(j < cnt, h2, 0.0)
        return jnp.maximum(pooled, h2)

    pooled = jax.lax.fori_loop(
        0, _NSAMPLE, slot, jnp.zeros((_PBLK, _COUT), jnp.float32))
    out_ref[0] = pooled


def kernel(xyz, features, W1, b1, W2, b2):
    xyz_t = jnp.transpose(xyz, (0, 2, 1))
    xyz_r = xyz_t.reshape(_B, 3, _ROWS, _LANES)

    pos_grid = jnp.asarray(
        np.arange(_N, dtype=np.int32).reshape(_ROWS, _LANES))
    new_xyz = pl.pallas_call(
        _fps_kernel,
        out_shape=jax.ShapeDtypeStruct((_B, _NPOINT, 3), jnp.float32),
        scratch_shapes=[pltpu.VMEM((_B, _ROWS, _LANES), jnp.float32)],
    )(xyz_r, xyz, pos_grid)

    src = jnp.concatenate([xyz, jnp.transpose(features, (0, 2, 1))], axis=-1)

    out = pl.pallas_call(
        _group_mlp_kernel,
        grid=(_B, _NPOINT // _PBLK),
        in_specs=[
            pl.BlockSpec((1, 3, _N), lambda b, p: (b, 0, 0)),
            pl.BlockSpec((1, _PBLK, 3), lambda b, p: (b, p, 0)),
            pl.BlockSpec((1, _N, _CIN), lambda b, p: (b, 0, 0)),
            pl.BlockSpec((_CIN, _CMID), lambda b, p: (0, 0)),
            pl.BlockSpec((1, _CMID), lambda b, p: (0, 0)),
            pl.BlockSpec((_CMID, _COUT), lambda b, p: (0, 0)),
            pl.BlockSpec((1, _COUT), lambda b, p: (0, 0)),
        ],
        out_specs=pl.BlockSpec((1, _PBLK, _COUT), lambda b, p: (b, p, 0)),
        out_shape=jax.ShapeDtypeStruct((_B, _NPOINT, _COUT), jnp.float32),
    )(xyz_t, new_xyz, src, W1.T, b1[None, :], W2.T, b2[None, :])

    new_features = jnp.transpose(out, (0, 2, 1))
    return (new_xyz, new_features)


# speedup vs baseline: 1.1054x; 1.1054x over previous
import numpy as np
import jax
import jax.numpy as jnp
from jax.experimental import pallas as pl
from jax.experimental.pallas import tpu as pltpu

_B, _N, _C = 2, 8192, 32
_NPOINT = 1024
_NSAMPLE = 32
_RADIUS2 = np.float32(0.8 * 0.8)
_CIN = _C + 3
_CMID, _COUT = 64, 64
_ROWS, _LANES = 64, 128
_PROWS = _NPOINT // _LANES
_PBLK = 128

_HIGHEST = jax.lax.Precision.HIGHEST


_SL = 8
_NSLICE = _ROWS // _SL


def _fps_kernel(xyz_ref, xyzrow_ref, pos_ref, out_ref, dists_ref):
    def lexmax(a, b):
        take_b = jnp.logical_or(
            b[0] > a[0], jnp.logical_and(b[0] == a[0], b[1] < a[1]))
        return tuple(jnp.where(take_b, bb, aa) for aa, bb in zip(a, b))

    def lane_phase(w, shifts):
        rolled = [tuple(pltpu.roll(t, sh, 1) for t in w) for sh in shifts]
        for r in rolled:
            w = lexmax(w, r)
        return w

    def body(i, st):
        ws = []
        for b in range(_B):
            cx, cy, cz = st[b]
            out_ref[b, pl.ds(i, 1), :] = jnp.concatenate(
                [cx[0:1, 0:1], cy[0:1, 0:1], cz[0:1, 0:1]], axis=1)
            w = None
            for j in range(_NSLICE):
                sl = slice(j * _SL, (j + 1) * _SL)
                xj = xyz_ref[b, 0, sl, :]
                yj = xyz_ref[b, 1, sl, :]
                zj = xyz_ref[b, 2, sl, :]
                dx = xj - cx
                dy = yj - cy
                dz = zj - cz
                d = dx * dx + dy * dy + dz * dz
                dj = jnp.minimum(dists_ref[b, sl, :], d)
                dists_ref[b, sl, :] = dj
                cand = (dj, pos_ref[sl, :], xj, yj, zj)
                w = cand if w is None else lexmax(w, cand)
            for sh in (4, 2, 1):
                w = lexmax(w, tuple(pltpu.roll(t, sh, 0) for t in w))
            ws.append(w)
        ws = [lane_phase(w, (1, 2, 3)) for w in ws]
        ws = [lane_phase(w, (4, 8, 12)) for w in ws]
        ws = [lane_phase(w, (16, 32, 48)) for w in ws]
        ws = [lane_phase(w, (64,)) for w in ws]
        return tuple((w[2], w[3], w[4]) for w in ws)

    dists_ref[...] = jnp.full((_B, _ROWS, _LANES), 1e10, jnp.float32)
    init = []
    for b in range(_B):
        c0 = [jnp.full((_SL, _LANES), xyzrow_ref[b, 0, k], jnp.float32)
              for k in range(3)]
        init.append((c0[0], c0[1], c0[2]))
    jax.lax.fori_loop(0, _NPOINT, body, tuple(init))


def _group_mlp_kernel(xyzt_ref, nxyz_ref, src_ref, w1_ref, b1_ref,
                      w2_ref, b2_ref, out_ref):
    xs = xyzt_ref[0]
    cx = nxyz_ref[0]

    d2 = None
    for k in range(3):
        diff = cx[:, k:k + 1] - xs[k:k + 1, :]
        sq = diff * diff
        d2 = sq if d2 is None else d2 + sq
    mask = d2 < _RADIUS2
    mi = mask.astype(jnp.int32)

    c = mi
    sh = 1
    while sh < _N:
        c = c + jnp.concatenate(
            [jnp.zeros((_PBLK, sh), jnp.int32), c[:, :_N - sh]], axis=1)
        sh *= 2
    cnt = c[:, _N - 1:_N]
    key = c * mi

    srcf = src_ref[0]
    hi = srcf.astype(jnp.bfloat16)
    r1 = srcf - hi.astype(jnp.float32)
    lo = r1.astype(jnp.bfloat16)
    lo2 = (r1 - lo.astype(jnp.float32)).astype(jnp.bfloat16)

    nxyz_pad = jnp.concatenate(
        [cx, jnp.zeros((_PBLK, _CIN - 3), jnp.float32)], axis=1)
    w1 = w1_ref[...]
    b1 = b1_ref[...]
    w2 = w2_ref[...]
    b2 = b2_ref[...]

    def slot(j, pooled):
        eqb = (key == j + 1).astype(jnp.bfloat16)
        g = (jnp.dot(eqb, hi, preferred_element_type=jnp.float32)
             + jnp.dot(eqb, lo, preferred_element_type=jnp.float32)
             + jnp.dot(eqb, lo2, preferred_element_type=jnp.float32))
        u = g - nxyz_pad
        h1 = jnp.maximum(
            jnp.dot(u, w1, preferred_element_type=jnp.float32,
                    precision=_HIGHEST) + b1, 0.0)
        h2 = jnp.maximum(
            jnp.dot(h1, w2, preferred_element_type=jnp.float32,
                    precision=_HIGHEST) + b2, 0.0)
        h2 = jnp.where(j < cnt, h2, 0.0)
        return jnp.maximum(pooled, h2)

    pooled = jax.lax.fori_loop(
        0, _NSAMPLE, slot, jnp.zeros((_PBLK, _COUT), jnp.float32))
    out_ref[0] = pooled


def kernel(xyz, features, W1, b1, W2, b2):
    xyz_t = jnp.transpose(xyz, (0, 2, 1))
    xyz_r = xyz_t.reshape(_B, 3, _ROWS, _LANES)

    pos_grid = jnp.asarray(
        np.arange(_N, dtype=np.int32).reshape(_ROWS, _LANES))
    new_xyz = pl.pallas_call(
        _fps_kernel,
        out_shape=jax.ShapeDtypeStruct((_B, _NPOINT, 3), jnp.float32),
        scratch_shapes=[pltpu.VMEM((_B, _ROWS, _LANES), jnp.float32)],
    )(xyz_r, xyz, pos_grid)

    src = jnp.concatenate([xyz, jnp.transpose(features, (0, 2, 1))], axis=-1)

    out = pl.pallas_call(
        _group_mlp_kernel,
        grid=(_B, _NPOINT // _PBLK),
        in_specs=[
            pl.BlockSpec((1, 3, _N), lambda b, p: (b, 0, 0)),
            pl.BlockSpec((1, _PBLK, 3), lambda b, p: (b, p, 0)),
            pl.BlockSpec((1, _N, _CIN), lambda b, p: (b, 0, 0)),
            pl.BlockSpec((_CIN, _CMID), lambda b, p: (0, 0)),
            pl.BlockSpec((1, _CMID), lambda b, p: (0, 0)),
            pl.BlockSpec((_CMID, _COUT), lambda b, p: (0, 0)),
            pl.BlockSpec((1, _COUT), lambda b, p: (0, 0)),
        ],
        out_specs=pl.BlockSpec((1, _PBLK, _COUT), lambda b, p: (b, p, 0)),
        out_shape=jax.ShapeDtypeStruct((_B, _NPOINT, _COUT), jnp.float32),
    )(xyz_t, new_xyz, src, W1.T, b1[None, :], W2.T, b2[None, :])

    new_features = jnp.transpose(out, (0, 2, 1))
    return (new_xyz, new_features)


# speedup vs baseline: 1.3209x; 1.1949x over previous
import numpy as np
import jax
import jax.numpy as jnp
from jax.experimental import pallas as pl
from jax.experimental.pallas import tpu as pltpu

_B, _N, _C = 2, 8192, 32
_NPOINT = 1024
_NSAMPLE = 32
_RADIUS2 = np.float32(0.8 * 0.8)
_CIN = _C + 3
_CMID, _COUT = 64, 64
_ROWS, _LANES = 64, 128
_PROWS = _NPOINT // _LANES
_PBLK = 128

_HIGHEST = jax.lax.Precision.HIGHEST


_SL = 8
_NSLICE = _ROWS // _SL


def _fps_kernel(xyz_ref, xyzrow_ref, pos_ref, out_ref, dists_ref):
    def lexmax(a, b):
        take_b = jnp.logical_or(
            b[0] > a[0], jnp.logical_and(b[0] == a[0], b[1] < a[1]))
        return tuple(jnp.where(take_b, bb, aa) for aa, bb in zip(a, b))

    def lane_phase(w, shifts):
        rolled = [tuple(pltpu.roll(t, sh, 1) for t in w) for sh in shifts]
        for r in rolled:
            w = lexmax(w, r)
        return w

    def body(i, st):
        ws = []
        for b in range(_B):
            cx, cy, cz = st[b]
            out_ref[b, pl.ds(i, 1), :] = jnp.concatenate(
                [cx[0:1, 0:1], cy[0:1, 0:1], cz[0:1, 0:1]], axis=1)
            w = None
            for j in range(_NSLICE):
                sl = slice(j * _SL, (j + 1) * _SL)
                xj = xyz_ref[b, 0, sl, :]
                yj = xyz_ref[b, 1, sl, :]
                zj = xyz_ref[b, 2, sl, :]
                dx = xj - cx
                dy = yj - cy
                dz = zj - cz
                d = dx * dx + dy * dy + dz * dz
                dj = jnp.minimum(dists_ref[b, sl, :], d)
                dists_ref[b, sl, :] = dj
                cand = (dj, pos_ref[sl, :], xj, yj, zj)
                w = cand if w is None else lexmax(w, cand)
            for sh in (4, 2, 1):
                w = lexmax(w, tuple(pltpu.roll(t, sh, 0) for t in w))
            ws.append(w)
        ws = [lane_phase(w, (1, 2, 3)) for w in ws]
        ws = [lane_phase(w, (4, 8, 12)) for w in ws]
        ws = [lane_phase(w, (16, 32, 48)) for w in ws]
        ws = [lane_phase(w, (64,)) for w in ws]
        return tuple((w[2], w[3], w[4]) for w in ws)

    dists_ref[...] = jnp.full((_B, _ROWS, _LANES), 1e10, jnp.float32)
    init = []
    for b in range(_B):
        c0 = [jnp.full((_SL, _LANES), xyzrow_ref[b, 0, k], jnp.float32)
              for k in range(3)]
        init.append((c0[0], c0[1], c0[2]))
    jax.lax.fori_loop(0, _NPOINT, body, tuple(init))


def _group_mlp_kernel(xyzt_ref, nxyz_ref, src_ref, w1_ref, b1_ref,
                      w2_ref, b2_ref, out_ref):
    xs = xyzt_ref[0]
    cx = nxyz_ref[0]

    d2 = None
    for k in range(3):
        diff = cx[:, k:k + 1] - xs[k:k + 1, :]
        sq = diff * diff
        d2 = sq if d2 is None else d2 + sq
    mask = d2 < _RADIUS2
    mi = mask.astype(jnp.int32)

    c = mi
    sh = 1
    while sh < _N:
        c = c + jnp.concatenate(
            [jnp.zeros((_PBLK, sh), jnp.int32), c[:, :_N - sh]], axis=1)
        sh *= 2
    cnt = c[:, _N - 1:_N]
    key = c * mi

    srcf = src_ref[0]
    hi = srcf.astype(jnp.bfloat16)
    lo = (srcf - hi.astype(jnp.float32)).astype(jnp.bfloat16)

    nxyz_pad = jnp.concatenate(
        [cx, jnp.zeros((_PBLK, _CIN - 3), jnp.float32)], axis=1)
    w1 = w1_ref[...]
    b1 = b1_ref[...]
    w2 = w2_ref[...]
    b2 = b2_ref[...]

    def slot(j, pooled):
        eqb = (key == j + 1).astype(jnp.bfloat16)
        g = (jnp.dot(eqb, hi, preferred_element_type=jnp.float32)
             + jnp.dot(eqb, lo, preferred_element_type=jnp.float32))
        u = g - nxyz_pad
        h1 = jnp.maximum(
            jnp.dot(u, w1, preferred_element_type=jnp.float32,
                    precision=_HIGHEST) + b1, 0.0)
        h2 = jnp.maximum(
            jnp.dot(h1, w2, preferred_element_type=jnp.float32,
                    precision=_HIGHEST) + b2, 0.0)
        h2 = jnp.where(j < cnt, h2, 0.0)
        return jnp.maximum(pooled, h2)

    pooled = jax.lax.fori_loop(
        0, _NSAMPLE, slot, jnp.zeros((_PBLK, _COUT), jnp.float32))
    out_ref[0] = pooled


def kernel(xyz, features, W1, b1, W2, b2):
    xyz_t = jnp.transpose(xyz, (0, 2, 1))
    xyz_r = xyz_t.reshape(_B, 3, _ROWS, _LANES)

    pos_grid = jnp.asarray(
        np.arange(_N, dtype=np.int32).reshape(_ROWS, _LANES))
    new_xyz = pl.pallas_call(
        _fps_kernel,
        out_shape=jax.ShapeDtypeStruct((_B, _NPOINT, 3), jnp.float32),
        scratch_shapes=[pltpu.VMEM((_B, _ROWS, _LANES), jnp.float32)],
    )(xyz_r, xyz, pos_grid)

    src = jnp.concatenate([xyz, jnp.transpose(features, (0, 2, 1))], axis=-1)

    out = pl.pallas_call(
        _group_mlp_kernel,
        grid=(_B, _NPOINT // _PBLK),
        in_specs=[
            pl.BlockSpec((1, 3, _N), lambda b, p: (b, 0, 0)),
            pl.BlockSpec((1, _PBLK, 3), lambda b, p: (b, p, 0)),
            pl.BlockSpec((1, _N, _CIN), lambda b, p: (b, 0, 0)),
            pl.BlockSpec((_CIN, _CMID), lambda b, p: (0, 0)),
            pl.BlockSpec((1, _CMID), lambda b, p: (0, 0)),
            pl.BlockSpec((_CMID, _COUT), lambda b, p: (0, 0)),
            pl.BlockSpec((1, _COUT), lambda b, p: (0, 0)),
        ],
        out_specs=pl.BlockSpec((1, _PBLK, _COUT), lambda b, p: (b, p, 0)),
        out_shape=jax.ShapeDtypeStruct((_B, _NPOINT, _COUT), jnp.float32),
    )(xyz_t, new_xyz, src, W1.T, b1[None, :], W2.T, b2[None, :])

    new_features = jnp.transpose(out, (0, 2, 1))
    return (new_xyz, new_features)


# speedup vs baseline: 1.3237x; 1.0021x over previous
import numpy as np
import jax
import jax.numpy as jnp
from jax.experimental import pallas as pl
from jax.experimental.pallas import tpu as pltpu

_B, _N, _C = 2, 8192, 32
_NPOINT = 1024
_NSAMPLE = 32
_RADIUS2 = np.float32(0.8 * 0.8)
_CIN = _C + 3
_CMID, _COUT = 64, 64
_ROWS, _LANES = 64, 128
_PROWS = _NPOINT // _LANES
_PBLK = 128

_HIGH = jax.lax.Precision.HIGHEST


_SL = 8
_NSLICE = _ROWS // _SL


def _fps_kernel(xyz_ref, xyzrow_ref, pos_ref, out_ref, dists_ref):
    def lexmax(a, b):
        take_b = jnp.logical_or(
            b[0] > a[0], jnp.logical_and(b[0] == a[0], b[1] < a[1]))
        return tuple(jnp.where(take_b, bb, aa) for aa, bb in zip(a, b))

    def lane_phase(w, shifts):
        rolled = [tuple(pltpu.roll(t, sh, 1) for t in w) for sh in shifts]
        for r in rolled:
            w = lexmax(w, r)
        return w

    def body(i, st):
        ws = []
        for b in range(_B):
            cx, cy, cz = st[b]
            out_ref[b, pl.ds(i, 1), :] = jnp.concatenate(
                [cx[0:1, 0:1], cy[0:1, 0:1], cz[0:1, 0:1]], axis=1)
            w = None
            for j in range(_NSLICE):
                sl = slice(j * _SL, (j + 1) * _SL)
                xj = xyz_ref[b, 0, sl, :]
                yj = xyz_ref[b, 1, sl, :]
                zj = xyz_ref[b, 2, sl, :]
                dx = xj - cx
                dy = yj - cy
                dz = zj - cz
                d = dx * dx + dy * dy + dz * dz
                dj = jnp.minimum(dists_ref[b, sl, :], d)
                dists_ref[b, sl, :] = dj
                cand = (dj, pos_ref[sl, :], xj, yj, zj)
                w = cand if w is None else lexmax(w, cand)
            for sh in (4, 2, 1):
                w = lexmax(w, tuple(pltpu.roll(t, sh, 0) for t in w))
            ws.append(w)
        ws = [lane_phase(w, (1, 2, 3)) for w in ws]
        ws = [lane_phase(w, (4, 8, 12)) for w in ws]
        ws = [lane_phase(w, (16, 32, 48)) for w in ws]
        ws = [lane_phase(w, (64,)) for w in ws]
        return tuple((w[2], w[3], w[4]) for w in ws)

    dists_ref[...] = jnp.full((_B, _ROWS, _LANES), 1e10, jnp.float32)
    init = []
    for b in range(_B):
        c0 = [jnp.full((_SL, _LANES), xyzrow_ref[b, 0, k], jnp.float32)
              for k in range(3)]
        init.append((c0[0], c0[1], c0[2]))
    jax.lax.fori_loop(0, _NPOINT, body, tuple(init))


def _group_mlp_kernel(xyzt_ref, nxyz_ref, src_ref, w1_ref, b1_ref,
                      w2_ref, b2_ref, out_ref):
    xs = xyzt_ref[0]
    cx = nxyz_ref[0]

    d2 = None
    for k in range(3):
        diff = cx[:, k:k + 1] - xs[k:k + 1, :]
        sq = diff * diff
        d2 = sq if d2 is None else d2 + sq
    mask = d2 < _RADIUS2
    mi = mask.astype(jnp.int32)

    c = mi
    sh = 1
    while sh < _N:
        c = c + jnp.concatenate(
            [jnp.zeros((_PBLK, sh), jnp.int32), c[:, :_N - sh]], axis=1)
        sh *= 2
    cnt = c[:, _N - 1:_N]
    key = c * mi

    srcf = src_ref[0]
    hi = srcf.astype(jnp.bfloat16)
    lo = (srcf - hi.astype(jnp.float32)).astype(jnp.bfloat16)

    nxyz_pad = jnp.concatenate(
        [cx, jnp.zeros((_PBLK, _CIN - 3), jnp.float32)], axis=1)
    w1 = w1_ref[...]
    b1 = b1_ref[...]
    w2 = w2_ref[...]
    b2 = b2_ref[...]

    def slot(j, pooled):
        eqb = (key == j + 1).astype(jnp.bfloat16)
        g = (jnp.dot(eqb, hi, preferred_element_type=jnp.float32)
             + jnp.dot(eqb, lo, preferred_element_type=jnp.float32))
        u = g - nxyz_pad
        h1 = jnp.maximum(
            jnp.dot(u, w1, preferred_element_type=jnp.float32,
                    precision=_HIGH) + b1, 0.0)
        h2 = jnp.maximum(
            jnp.dot(h1, w2, preferred_element_type=jnp.float32,
                    precision=_HIGH) + b2, 0.0)
        h2 = jnp.where(j < cnt, h2, 0.0)
        return jnp.maximum(pooled, h2)

    pooled = jax.lax.fori_loop(
        0, _NSAMPLE, slot, jnp.zeros((_PBLK, _COUT), jnp.float32))
    out_ref[0] = pooled


def kernel(xyz, features, W1, b1, W2, b2):
    xyz_t = jnp.transpose(xyz, (0, 2, 1))
    xyz_r = xyz_t.reshape(_B, 3, _ROWS, _LANES)

    pos_grid = jnp.asarray(
        np.arange(_N, dtype=np.int32).reshape(_ROWS, _LANES))
    new_xyz = pl.pallas_call(
        _fps_kernel,
        out_shape=jax.ShapeDtypeStruct((_B, _NPOINT, 3), jnp.float32),
        scratch_shapes=[pltpu.VMEM((_B, _ROWS, _LANES), jnp.float32)],
    )(xyz_r, xyz, pos_grid)

    src = jnp.concatenate([xyz, jnp.transpose(features, (0, 2, 1))], axis=-1)

    out = pl.pallas_call(
        _group_mlp_kernel,
        grid=(_B, _NPOINT // _PBLK),
        in_specs=[
            pl.BlockSpec((1, 3, _N), lambda b, p: (b, 0, 0)),
            pl.BlockSpec((1, _PBLK, 3), lambda b, p: (b, p, 0)),
            pl.BlockSpec((1, _N, _CIN), lambda b, p: (b, 0, 0)),
            pl.BlockSpec((_CIN, _CMID), lambda b, p: (0, 0)),
            pl.BlockSpec((1, _CMID), lambda b, p: (0, 0)),
            pl.BlockSpec((_CMID, _COUT), lambda b, p: (0, 0)),
            pl.BlockSpec((1, _COUT), lambda b, p: (0, 0)),
        ],
        out_specs=pl.BlockSpec((1, _PBLK, _COUT), lambda b, p: (b, p, 0)),
        out_shape=jax.ShapeDtypeStruct((_B, _NPOINT, _COUT), jnp.float32),
    )(xyz_t, new_xyz, src, W1.T, b1[None, :], W2.T, b2[None, :])

    new_features = jnp.transpose(out, (0, 2, 1))
    return (new_xyz, new_features)
